# baseline (speedup 1.0000x reference)
import functools

import jax
import jax.numpy as jnp
from jax import lax
from jax.experimental import pallas as pl
from jax.experimental.pallas import tpu as pltpu
from jax.experimental.pallas import tpu_sc as plsc

N_AXES = 4
PER_AXIS = 32
NUM_WORKERS = 32
TOK_PER_WORKER = 1024
CHUNK_TOK = 256
N_CHUNKS = TOK_PER_WORKER // CHUNK_TOK
IDS_PER_ROW = 128
ROWS_PER_AXIS = CHUNK_TOK // IDS_PER_ROW
NBUF = 2


def _embed_body(ids_hbm, w0, w1, w2, w3, out_hbm, raw_v, idx_v, rows_v, sems):
    tables = (w0, w1, w2, w3)
    wid = lax.axis_index("s") * 2 + lax.axis_index("c")
    seq = ids_hbm.shape[1]
    wpb = seq // TOK_PER_WORKER
    b = wid // wpb
    s0 = (wid % wpb) * TOK_PER_WORKER
    lane = lax.iota(jnp.int32, 16)

    def load_chunk(c, buf):
        pltpu.sync_copy(ids_hbm.at[b, pl.ds(s0 + c * CHUNK_TOK, CHUNK_TOK)],
                        raw_v.at[buf])
        for i in range(N_AXES):
            ax = jnp.full((16,), i, jnp.int32)
            for g in range(CHUNK_TOK // 16):
                v = plsc.load_gather(raw_v.at[buf], [lane + g * 16, ax])
                j, col = (g * 16) // IDS_PER_ROW, (g * 16) % IDS_PER_ROW
                idx_v[buf, i, j, pl.ds(col, 16)] = v
        for i in range(N_AXES):
            for j in range(ROWS_PER_AXIS):
                pltpu.make_async_copy(
                    tables[i].at[idx_v.at[buf, i, j]],
                    rows_v.at[buf, i, pl.ds(j * IDS_PER_ROW, IDS_PER_ROW)],
                    sems.at[buf],
                ).start()

    def drain_chunk(c, buf):
        for i in range(N_AXES):
            for j in range(ROWS_PER_AXIS):
                pltpu.make_async_copy(
                    tables[i].at[idx_v.at[buf, i, j]],
                    rows_v.at[buf, i, pl.ds(j * IDS_PER_ROW, IDS_PER_ROW)],
                    sems.at[buf],
                ).wait()
        for i in range(N_AXES):
            pltpu.sync_copy(
                rows_v.at[buf, i],
                out_hbm.at[b, 0, pl.ds(s0 + c * CHUNK_TOK, CHUNK_TOK),
                           pl.ds(i * PER_AXIS, PER_AXIS)])

    load_chunk(0, 0)
    for c in range(N_CHUNKS):
        if c + 1 < N_CHUNKS:
            load_chunk(c + 1, (c + 1) % NBUF)
        drain_chunk(c, c % NBUF)


def kernel(ids, W0, W1, W2, W3):
    batch, seq, n_axes = ids.shape

    mesh = plsc.VectorSubcoreMesh(core_axis_name="c", subcore_axis_name="s")
    run = functools.partial(
        pl.kernel,
        out_type=jax.ShapeDtypeStruct((batch, 1, seq, N_AXES * PER_AXIS),
                                      jnp.float32),
        mesh=mesh,
        scratch_types=[
            pltpu.VMEM((NBUF, CHUNK_TOK, N_AXES), jnp.int32),
            pltpu.VMEM((NBUF, N_AXES, ROWS_PER_AXIS, IDS_PER_ROW), jnp.int32),
            pltpu.VMEM((NBUF, N_AXES, CHUNK_TOK, PER_AXIS), jnp.float32),
            pltpu.SemaphoreType.DMA((NBUF,)),
        ],
        compiler_params=pltpu.CompilerParams(
            use_tc_tiling_on_sc=False, needs_layout_passes=False),
    )(_embed_body)
    return run(ids.astype(jnp.int32), W0, W1, W2, W3)


# speedup vs baseline: 1.5901x; 1.5901x over previous
import functools

import jax
import jax.numpy as jnp
from jax import lax
from jax.experimental import pallas as pl
from jax.experimental.pallas import tpu as pltpu
from jax.experimental.pallas import tpu_sc as plsc

N_AXES = 4
PER_AXIS = 32
NUM_WORKERS = 32
TOK_PER_WORKER = 1024
CHUNK_TOK = 256
N_CHUNKS = TOK_PER_WORKER // CHUNK_TOK
IDS_PER_ROW = 128
ROWS_PER_AXIS = CHUNK_TOK // IDS_PER_ROW
NBUF = 2


def _embed_body(ids_hbm, w0, w1, w2, w3, out_hbm, idx_v, rows_v, sems):
    tables = (w0, w1, w2, w3)
    wid = lax.axis_index("s") * 2 + lax.axis_index("c")
    seq = out_hbm.shape[2]
    wpb = seq // TOK_PER_WORKER
    b = wid // wpb
    s0 = (wid % wpb) * TOK_PER_WORKER

    def load_chunk(c, buf):
        blk0 = s0 // IDS_PER_ROW + c * ROWS_PER_AXIS
        for i in range(N_AXES):
            pltpu.sync_copy(
                ids_hbm.at[b * N_AXES + i, pl.ds(blk0, ROWS_PER_AXIS)],
                idx_v.at[buf, i])
        for i in range(N_AXES):
            for j in range(ROWS_PER_AXIS):
                pltpu.make_async_copy(
                    tables[i].at[idx_v.at[buf, i, j]],
                    rows_v.at[buf, i, pl.ds(j * IDS_PER_ROW, IDS_PER_ROW)],
                    sems.at[buf],
                ).start()

    def drain_chunk(c, buf):
        for i in range(N_AXES):
            for j in range(ROWS_PER_AXIS):
                pltpu.make_async_copy(
                    tables[i].at[idx_v.at[buf, i, j]],
                    rows_v.at[buf, i, pl.ds(j * IDS_PER_ROW, IDS_PER_ROW)],
                    sems.at[buf],
                ).wait()
        for i in range(N_AXES):
            pltpu.sync_copy(
                rows_v.at[buf, i],
                out_hbm.at[b, 0, pl.ds(s0 + c * CHUNK_TOK, CHUNK_TOK),
                           pl.ds(i * PER_AXIS, PER_AXIS)])

    load_chunk(0, 0)
    for c in range(N_CHUNKS):
        if c + 1 < N_CHUNKS:
            load_chunk(c + 1, (c + 1) % NBUF)
        drain_chunk(c, c % NBUF)


def kernel(ids, W0, W1, W2, W3):
    batch, seq, n_axes = ids.shape
    ids_t = ids.astype(jnp.int32).transpose(0, 2, 1).reshape(
        batch * n_axes, seq // IDS_PER_ROW, IDS_PER_ROW)

    mesh = plsc.VectorSubcoreMesh(core_axis_name="c", subcore_axis_name="s")
    run = functools.partial(
        pl.kernel,
        out_type=jax.ShapeDtypeStruct((batch, 1, seq, N_AXES * PER_AXIS),
                                      jnp.float32),
        mesh=mesh,
        scratch_types=[
            pltpu.VMEM((NBUF, N_AXES, ROWS_PER_AXIS, IDS_PER_ROW), jnp.int32),
            pltpu.VMEM((NBUF, N_AXES, CHUNK_TOK, PER_AXIS), jnp.float32),
            pltpu.SemaphoreType.DMA((NBUF,)),
        ],
        compiler_params=pltpu.CompilerParams(
            use_tc_tiling_on_sc=False, needs_layout_passes=False),
    )(_embed_body)
    return run(ids_t, W0, W1, W2, W3)
